# baseline (speedup 1.0000x reference)
import functools

import jax
import jax.numpy as jnp
import numpy as np
from jax import lax
from jax.experimental import pallas as pl
from jax.experimental.pallas import tpu as pltpu
from jax.experimental.pallas import tpu_sc as plsc

N_UNIQ = 100000
STATE_SIZE = 128
HALF = 8192
NUM_LEFT = N_UNIQ - HALF

V = 1024
C = 512
B = 200
N_PAD = B * C
PAD_VAL = 1023

NW = 32
LROW = 128
KA = N_PAD // NW // LROW
KB = (2 * HALF) // NW // LROW
TRASH = 2 * HALF
SRC_PAD = TRASH + LROW


def _np_threefry2x32(k1, k2, x0, x1):
    def rotl(x, d):
        return ((x << np.uint32(d)) | (x >> np.uint32(32 - d))).astype(np.uint32)
    ks = [np.uint32(k1), np.uint32(k2),
          np.uint32(k1) ^ np.uint32(k2) ^ np.uint32(0x1BD11BDA)]
    rots = [(13, 15, 26, 6), (17, 29, 16, 24)]
    x0 = (x0 + ks[0]).astype(np.uint32)
    x1 = (x1 + ks[1]).astype(np.uint32)
    for i in range(5):
        for r in rots[i % 2]:
            x0 = (x0 + x1).astype(np.uint32)
            x1 = rotl(x1, r)
            x1 = x0 ^ x1
        x0 = (x0 + ks[(i + 1) % 3]).astype(np.uint32)
        x1 = (x1 + ks[(i + 2) % 3] + np.uint32(i + 1)).astype(np.uint32)
    return x0, x1


def _np_permutation(seed, n):
    key = np.array([np.uint32(np.uint64(seed) >> np.uint64(32)),
                    np.uint32(np.uint64(seed) & np.uint64(0xFFFFFFFF))])
    x = np.arange(n, dtype=np.int32)
    num_rounds = int(np.ceil(3 * np.log(max(1, n)) / np.log(2**32 - 1)))
    for _ in range(num_rounds):
        b1, b2 = _np_threefry2x32(key[0], key[1],
                                  np.zeros(2, np.uint32),
                                  np.arange(2, dtype=np.uint32))
        key, subkey = np.stack([b1, b2], 1)
        s1, s2 = _np_threefry2x32(subkey[0], subkey[1],
                                  np.zeros(n, np.uint32),
                                  np.arange(n, dtype=np.uint32))
        x = x[np.argsort(s1 ^ s2, kind="stable")]
    return x


_PERM = _np_permutation(12345, NUM_LEFT)[:HALF]
_SLOT_OF_POS = np.full((N_PAD,), TRASH, dtype=np.int32)
_SLOT_OF_POS[NUM_LEFT:N_UNIQ] = np.arange(HALF, dtype=np.int32)
_SLOT_OF_POS[_PERM] = np.arange(HALF, 2 * HALF, dtype=np.int32)
_IDS = np.arange(N_PAD, dtype=np.int32)


def _tc_rank_body(row_ref, col_ref, rin_ref, key_ref, hist_ref):
    b = pl.program_id(0)
    crow = row_ref[...].reshape(1, C)
    ccol = col_ref[...].reshape(C, 1)
    iota_v = lax.broadcasted_iota(jnp.int32, (1, V), 1)
    h = (ccol == iota_v).astype(jnp.int32)
    hist_ref[...] = jnp.sum(h, axis=0, keepdims=True).astype(
        jnp.float32).reshape(1, 1, V)
    ii = lax.broadcasted_iota(jnp.int32, (C, C), 0)
    jj = lax.broadcasted_iota(jnp.int32, (C, C), 1)
    eq = ((ccol == crow) & (ii < jj)).astype(jnp.int32)
    rin_ref[...] = jnp.sum(eq, axis=0, keepdims=True).reshape(1, 1, C)
    key_ref[...] = row_ref[...] + b * V


def _tc_offs_body(hist_ref, out_ref):
    hist = hist_ref[...]
    tri = (lax.broadcasted_iota(jnp.int32, (B, B), 1)
           < lax.broadcasted_iota(jnp.int32, (B, B), 0)).astype(jnp.float32)
    colcum = jnp.dot(tri, hist, preferred_element_type=jnp.float32)
    total = jnp.sum(hist, axis=0, keepdims=True)
    mv = (lax.broadcasted_iota(jnp.int32, (V, V), 0)
          < lax.broadcasted_iota(jnp.int32, (V, V), 1)).astype(jnp.float32)
    glob = jnp.dot(total, mv, preferred_element_type=jnp.float32)
    out_ref[...] = (colcum + glob).astype(jnp.int32)


def _sc_scatter_body(key_hbm, rin_hbm, cnt_hbm, ids_hbm, choff_hbm,
                     slot_map_hbm, src_out, cnt_out,
                     key_v, rin_v, cnt_v, ids_v, off_v, slot_v, sem):
    wid = lax.axis_index("s") * 2 + lax.axis_index("c")
    ept = KA * LROW
    base = wid * ept
    pltpu.sync_copy(key_hbm.at[pl.ds(base, ept)], key_v)
    pltpu.sync_copy(rin_hbm.at[pl.ds(base, ept)], rin_v)
    pltpu.sync_copy(cnt_hbm.at[pl.ds(base, ept)], cnt_v)
    pltpu.sync_copy(ids_hbm.at[pl.ds(base, ept)], ids_v)
    cps = [pltpu.async_copy(choff_hbm.at[key_v.at[pl.ds(j * LROW, LROW)]],
                            off_v.at[pl.ds(j * LROW, LROW)], sem)
           for j in range(KA)]
    for cp in cps:
        cp.wait()
    for g in range(ept // 16):
        sl = pl.ds(g * 16, 16)
        rin_v[sl] = rin_v[sl] + off_v[sl]
    cps = [pltpu.async_copy(slot_map_hbm.at[rin_v.at[pl.ds(j * LROW, LROW)]],
                            slot_v.at[j], sem)
           for j in range(KA)]
    for cp in cps:
        cp.wait()
    cps = [pltpu.async_copy(ids_v.at[pl.ds(j * LROW, LROW)],
                            src_out.at[slot_v.at[j]], sem)
           for j in range(KA)]
    cps += [pltpu.async_copy(cnt_v.at[pl.ds(j * LROW, LROW)],
                             cnt_out.at[slot_v.at[j]], sem)
            for j in range(KA)]
    for cp in cps:
        cp.wait()


def _sc_gather_body(idx_hbm, samp_hbm, out_hbm, idx_v, rows_v, sem):
    wid = lax.axis_index("s") * 2 + lax.axis_index("c")
    base = wid * KB * LROW
    pltpu.sync_copy(idx_hbm.at[pl.ds(base, KB * LROW)], idx_v)
    for j in range(KB):
        pltpu.async_copy(samp_hbm.at[idx_v.at[pl.ds(j * LROW, LROW)]],
                         rows_v, sem).wait()
        pltpu.sync_copy(rows_v, out_hbm.at[pl.ds(base + j * LROW, LROW)])


def kernel(uniq_samples, uniq_count):
    i32 = jnp.int32
    cpad = jnp.concatenate(
        [uniq_count, jnp.full((N_PAD - N_UNIQ,), PAD_VAL, dtype=i32)])
    row3 = cpad.reshape(B, 1, C)
    col3 = cpad.reshape(B, C, 1)

    rin3, key3, hist3 = pl.pallas_call(
        _tc_rank_body,
        grid=(B,),
        in_specs=[
            pl.BlockSpec((1, 1, C), lambda b: (b, 0, 0)),
            pl.BlockSpec((1, C, 1), lambda b: (b, 0, 0)),
        ],
        out_specs=[
            pl.BlockSpec((1, 1, C), lambda b: (b, 0, 0)),
            pl.BlockSpec((1, 1, C), lambda b: (b, 0, 0)),
            pl.BlockSpec((1, 1, V), lambda b: (b, 0, 0)),
        ],
        out_shape=[
            jax.ShapeDtypeStruct((B, 1, C), i32),
            jax.ShapeDtypeStruct((B, 1, C), i32),
            jax.ShapeDtypeStruct((B, 1, V), jnp.float32),
        ],
    )(row3, col3)

    choff = pl.pallas_call(
        _tc_offs_body,
        out_shape=jax.ShapeDtypeStruct((B, V), i32),
    )(hist3.reshape(B, V))

    mesh = plsc.VectorSubcoreMesh(
        core_axis_name="c", subcore_axis_name="s",
        num_cores=2, num_subcores=16)

    sc_scatter = functools.partial(
        pl.kernel,
        out_type=(
            jax.ShapeDtypeStruct((SRC_PAD,), i32),
            jax.ShapeDtypeStruct((SRC_PAD,), i32),
        ),
        mesh=mesh,
        scratch_types=[pltpu.VMEM((KA * LROW,), i32) for _ in range(5)]
        + [pltpu.VMEM((KA, LROW), i32), pltpu.SemaphoreType.DMA],
    )(_sc_scatter_body)

    src_ext, cnt_ext = sc_scatter(
        key3.reshape(-1),
        rin3.reshape(-1),
        cpad,
        jnp.asarray(_IDS),
        choff.reshape(-1),
        jnp.asarray(_SLOT_OF_POS),
    )

    sc_gather = functools.partial(
        pl.kernel,
        out_type=jax.ShapeDtypeStruct((2 * HALF, STATE_SIZE), jnp.float32),
        mesh=mesh,
        scratch_types=[
            pltpu.VMEM((KB * LROW,), i32),
            pltpu.VMEM((LROW, STATE_SIZE), jnp.float32),
            pltpu.SemaphoreType.DMA,
        ],
    )(_sc_gather_body)

    out_samples = sc_gather(src_ext[: 2 * HALF], uniq_samples)
    out_counts = cnt_ext[: 2 * HALF]
    return out_samples, out_counts


# speedup vs baseline: 48.9688x; 48.9688x over previous
import functools

import jax
import jax.numpy as jnp
import numpy as np
from jax import lax
from jax.experimental import pallas as pl
from jax.experimental.pallas import tpu as pltpu
from jax.experimental.pallas import tpu_sc as plsc

N_UNIQ = 100000
STATE_SIZE = 128
HALF = 8192
NUM_LEFT = N_UNIQ - HALF

V = 1024
C = 512
B = 200
N_PAD = B * C
PAD_VAL = 1023

NW = 32
LROW = 128
KA = N_PAD // NW // LROW
KB = (2 * HALF) // NW // LROW
SRC_PAD = 2 * HALF + N_PAD


def _np_threefry2x32(k1, k2, x0, x1):
    def rotl(x, d):
        return ((x << np.uint32(d)) | (x >> np.uint32(32 - d))).astype(np.uint32)
    ks = [np.uint32(k1), np.uint32(k2),
          np.uint32(k1) ^ np.uint32(k2) ^ np.uint32(0x1BD11BDA)]
    rots = [(13, 15, 26, 6), (17, 29, 16, 24)]
    x0 = (x0 + ks[0]).astype(np.uint32)
    x1 = (x1 + ks[1]).astype(np.uint32)
    for i in range(5):
        for r in rots[i % 2]:
            x0 = (x0 + x1).astype(np.uint32)
            x1 = rotl(x1, r)
            x1 = x0 ^ x1
        x0 = (x0 + ks[(i + 1) % 3]).astype(np.uint32)
        x1 = (x1 + ks[(i + 2) % 3] + np.uint32(i + 1)).astype(np.uint32)
    return x0, x1


def _np_permutation(seed, n):
    key = np.array([np.uint32(np.uint64(seed) >> np.uint64(32)),
                    np.uint32(np.uint64(seed) & np.uint64(0xFFFFFFFF))])
    x = np.arange(n, dtype=np.int32)
    num_rounds = int(np.ceil(3 * np.log(max(1, n)) / np.log(2**32 - 1)))
    for _ in range(num_rounds):
        b1, b2 = _np_threefry2x32(key[0], key[1],
                                  np.zeros(2, np.uint32),
                                  np.arange(2, dtype=np.uint32))
        key, subkey = np.stack([b1, b2], 1)
        s1, s2 = _np_threefry2x32(subkey[0], subkey[1],
                                  np.zeros(n, np.uint32),
                                  np.arange(n, dtype=np.uint32))
        x = x[np.argsort(s1 ^ s2, kind="stable")]
    return x


_PERM = _np_permutation(12345, NUM_LEFT)[:HALF]
_SLOT_OF_POS = 2 * HALF + np.arange(N_PAD, dtype=np.int32)
_SLOT_OF_POS[NUM_LEFT:N_UNIQ] = np.arange(HALF, dtype=np.int32)
_SLOT_OF_POS[_PERM] = np.arange(HALF, 2 * HALF, dtype=np.int32)
_IDS = np.arange(N_PAD, dtype=np.int32)


def _tc_rank_body(row_ref, col_ref, rin_ref, key_ref, hist_ref):
    b = pl.program_id(0)
    crow = row_ref[...].reshape(1, C)
    ccol = col_ref[...].reshape(C, 1)
    iota_v = lax.broadcasted_iota(jnp.int32, (1, V), 1)
    h = (ccol == iota_v).astype(jnp.int32)
    hist_ref[...] = jnp.sum(h, axis=0, keepdims=True).astype(
        jnp.float32).reshape(1, 1, V)
    ii = lax.broadcasted_iota(jnp.int32, (C, C), 0)
    jj = lax.broadcasted_iota(jnp.int32, (C, C), 1)
    eq = ((ccol == crow) & (ii < jj)).astype(jnp.int32)
    rin_ref[...] = jnp.sum(eq, axis=0, keepdims=True).reshape(1, 1, C)
    key_ref[...] = row_ref[...] + b * V


def _tc_offs_body(hist_ref, out_ref):
    hist = hist_ref[...]
    tri = (lax.broadcasted_iota(jnp.int32, (B, B), 1)
           < lax.broadcasted_iota(jnp.int32, (B, B), 0)).astype(jnp.float32)
    colcum = jnp.dot(tri, hist, preferred_element_type=jnp.float32)
    total = jnp.sum(hist, axis=0, keepdims=True)
    mv = (lax.broadcasted_iota(jnp.int32, (V, V), 0)
          < lax.broadcasted_iota(jnp.int32, (V, V), 1)).astype(jnp.float32)
    glob = jnp.dot(total, mv, preferred_element_type=jnp.float32)
    out_ref[...] = (colcum + glob).astype(jnp.int32)


def _sc_scatter_body(key_hbm, rin_hbm, cnt_hbm, ids_hbm, choff_hbm,
                     slot_map_hbm, src_out, cnt_out,
                     key_v, rin_v, cnt_v, ids_v, off_v, slot_v, sem):
    wid = lax.axis_index("s") * 2 + lax.axis_index("c")
    ept = KA * LROW
    base = wid * ept
    pltpu.sync_copy(key_hbm.at[pl.ds(base, ept)], key_v)
    pltpu.sync_copy(rin_hbm.at[pl.ds(base, ept)], rin_v)
    pltpu.sync_copy(cnt_hbm.at[pl.ds(base, ept)], cnt_v)
    pltpu.sync_copy(ids_hbm.at[pl.ds(base, ept)], ids_v)
    cps = [pltpu.async_copy(choff_hbm.at[key_v.at[pl.ds(j * LROW, LROW)]],
                            off_v.at[pl.ds(j * LROW, LROW)], sem)
           for j in range(KA)]
    for cp in cps:
        cp.wait()
    for g in range(ept // 16):
        sl = pl.ds(g * 16, 16)
        rin_v[sl] = rin_v[sl] + off_v[sl]
    cps = [pltpu.async_copy(slot_map_hbm.at[rin_v.at[pl.ds(j * LROW, LROW)]],
                            slot_v.at[j], sem)
           for j in range(KA)]
    for cp in cps:
        cp.wait()
    cps = [pltpu.async_copy(ids_v.at[pl.ds(j * LROW, LROW)],
                            src_out.at[slot_v.at[j]], sem)
           for j in range(KA)]
    cps += [pltpu.async_copy(cnt_v.at[pl.ds(j * LROW, LROW)],
                             cnt_out.at[slot_v.at[j]], sem)
            for j in range(KA)]
    for cp in cps:
        cp.wait()


def _sc_gather_body(idx_hbm, samp_hbm, out_hbm, idx_v, rows_v, sem):
    wid = lax.axis_index("s") * 2 + lax.axis_index("c")
    base = wid * KB * LROW
    pltpu.sync_copy(idx_hbm.at[pl.ds(base, KB * LROW)], idx_v)
    for j in range(KB):
        pltpu.async_copy(samp_hbm.at[idx_v.at[pl.ds(j * LROW, LROW)]],
                         rows_v, sem).wait()
        pltpu.sync_copy(rows_v, out_hbm.at[pl.ds(base + j * LROW, LROW)])


def kernel(uniq_samples, uniq_count):
    i32 = jnp.int32
    cpad = jnp.concatenate(
        [uniq_count, jnp.full((N_PAD - N_UNIQ,), PAD_VAL, dtype=i32)])
    row3 = cpad.reshape(B, 1, C)
    col3 = cpad.reshape(B, C, 1)

    rin3, key3, hist3 = pl.pallas_call(
        _tc_rank_body,
        grid=(B,),
        in_specs=[
            pl.BlockSpec((1, 1, C), lambda b: (b, 0, 0)),
            pl.BlockSpec((1, C, 1), lambda b: (b, 0, 0)),
        ],
        out_specs=[
            pl.BlockSpec((1, 1, C), lambda b: (b, 0, 0)),
            pl.BlockSpec((1, 1, C), lambda b: (b, 0, 0)),
            pl.BlockSpec((1, 1, V), lambda b: (b, 0, 0)),
        ],
        out_shape=[
            jax.ShapeDtypeStruct((B, 1, C), i32),
            jax.ShapeDtypeStruct((B, 1, C), i32),
            jax.ShapeDtypeStruct((B, 1, V), jnp.float32),
        ],
    )(row3, col3)

    choff = pl.pallas_call(
        _tc_offs_body,
        out_shape=jax.ShapeDtypeStruct((B, V), i32),
    )(hist3.reshape(B, V))

    mesh = plsc.VectorSubcoreMesh(
        core_axis_name="c", subcore_axis_name="s",
        num_cores=2, num_subcores=16)

    sc_scatter = functools.partial(
        pl.kernel,
        out_type=(
            jax.ShapeDtypeStruct((SRC_PAD,), i32),
            jax.ShapeDtypeStruct((SRC_PAD,), i32),
        ),
        mesh=mesh,
        scratch_types=[pltpu.VMEM((KA * LROW,), i32) for _ in range(5)]
        + [pltpu.VMEM((KA, LROW), i32), pltpu.SemaphoreType.DMA],
    )(_sc_scatter_body)

    src_ext, cnt_ext = sc_scatter(
        key3.reshape(-1),
        rin3.reshape(-1),
        cpad,
        jnp.asarray(_IDS),
        choff.reshape(-1),
        jnp.asarray(_SLOT_OF_POS),
    )

    sc_gather = functools.partial(
        pl.kernel,
        out_type=jax.ShapeDtypeStruct((2 * HALF, STATE_SIZE), jnp.float32),
        mesh=mesh,
        scratch_types=[
            pltpu.VMEM((KB * LROW,), i32),
            pltpu.VMEM((LROW, STATE_SIZE), jnp.float32),
            pltpu.SemaphoreType.DMA,
        ],
    )(_sc_gather_body)

    out_samples = sc_gather(src_ext[: 2 * HALF], uniq_samples)
    out_counts = cnt_ext[: 2 * HALF]
    return out_samples, out_counts


# speedup vs baseline: 100.9272x; 2.0611x over previous
import functools

import jax
import jax.numpy as jnp
import numpy as np
from jax import lax
from jax.experimental import pallas as pl
from jax.experimental.pallas import tpu as pltpu
from jax.experimental.pallas import tpu_sc as plsc

N_UNIQ = 100000
STATE_SIZE = 128
HALF = 8192
NUM_LEFT = N_UNIQ - HALF

V = 1024
C = 128
B = 800
N_PAD = B * C
PAD_VAL = 1023
CPB = 8
TC_STEPS = B // CPB

NW = 32
LROW = 128
CPW = B // NW
EPT = CPW * LROW
KB = (2 * HALF) // NW


def _np_threefry2x32(k1, k2, x0, x1):
    def rotl(x, d):
        return ((x << np.uint32(d)) | (x >> np.uint32(32 - d))).astype(np.uint32)
    ks = [np.uint32(k1), np.uint32(k2),
          np.uint32(k1) ^ np.uint32(k2) ^ np.uint32(0x1BD11BDA)]
    rots = [(13, 15, 26, 6), (17, 29, 16, 24)]
    x0 = (x0 + ks[0]).astype(np.uint32)
    x1 = (x1 + ks[1]).astype(np.uint32)
    for i in range(5):
        for r in rots[i % 2]:
            x0 = (x0 + x1).astype(np.uint32)
            x1 = rotl(x1, r)
            x1 = x0 ^ x1
        x0 = (x0 + ks[(i + 1) % 3]).astype(np.uint32)
        x1 = (x1 + ks[(i + 2) % 3] + np.uint32(i + 1)).astype(np.uint32)
    return x0, x1


def _np_permutation(seed, n):
    key = np.array([np.uint32(np.uint64(seed) >> np.uint64(32)),
                    np.uint32(np.uint64(seed) & np.uint64(0xFFFFFFFF))])
    x = np.arange(n, dtype=np.int32)
    num_rounds = int(np.ceil(3 * np.log(max(1, n)) / np.log(2**32 - 1)))
    for _ in range(num_rounds):
        b1, b2 = _np_threefry2x32(key[0], key[1],
                                  np.zeros(2, np.uint32),
                                  np.arange(2, dtype=np.uint32))
        key, subkey = np.stack([b1, b2], 1)
        s1, s2 = _np_threefry2x32(subkey[0], subkey[1],
                                  np.zeros(n, np.uint32),
                                  np.arange(n, dtype=np.uint32))
        x = x[np.argsort(s1 ^ s2, kind="stable")]
    return x


_PERM = _np_permutation(12345, NUM_LEFT)[:HALF]
_POS_TAB = np.concatenate([NUM_LEFT + np.arange(HALF, dtype=np.int32),
                           _PERM.astype(np.int32)])
_IDS = np.arange(N_PAD, dtype=np.int32)


def _tc_rank_body(cnt_ref, rin_ref, key_ref):
    b = pl.program_id(0)
    cc = cnt_ref[...]
    ci = cc.reshape(CPB, C, 1)
    cj = cc.reshape(CPB, 1, C)
    ii = lax.broadcasted_iota(jnp.int32, (CPB, C, C), 1)
    jj = lax.broadcasted_iota(jnp.int32, (CPB, C, C), 2)
    eq = ((ci == cj) & (ii < jj)).astype(jnp.int32)
    rin_ref[...] = jnp.sum(eq, axis=1)
    chunk = b * CPB + lax.broadcasted_iota(jnp.int32, (CPB, C), 0)
    key_ref[...] = cc + chunk * V


def _tc_offs_body(hist_ref, out_ref):
    hist = hist_ref[...].astype(jnp.float32)
    tri = (lax.broadcasted_iota(jnp.int32, (B, B), 1)
           < lax.broadcasted_iota(jnp.int32, (B, B), 0)).astype(jnp.float32)
    colcum = jnp.dot(tri, hist, preferred_element_type=jnp.float32)
    total = jnp.sum(hist, axis=0, keepdims=True)
    mv = (lax.broadcasted_iota(jnp.int32, (V, V), 0)
          < lax.broadcasted_iota(jnp.int32, (V, V), 1)).astype(jnp.float32)
    glob = jnp.dot(total, mv, preferred_element_type=jnp.float32)
    out_ref[...] = (colcum + glob).astype(jnp.int32)


def _sc_hist_body(cnt_hbm, hist_out, cnt_v, hist_v):
    wid = lax.axis_index("s") * 2 + lax.axis_index("c")
    base = wid * EPT
    pltpu.sync_copy(cnt_hbm.at[pl.ds(base, EPT)], cnt_v)
    zeros16 = jnp.zeros((16,), jnp.int32)
    ones16 = jnp.ones((16,), jnp.int32)

    def zero_step(i, _):
        hist_v[pl.ds(i * 16, 16)] = zeros16
        return 0

    lax.fori_loop(0, CPW * V // 16, zero_step, 0)

    def acc_step(g, _):
        c16 = cnt_v[pl.ds(g * 16, 16)]
        idx = c16 + (g // (C // 16)) * V
        plsc.addupdate_scatter(hist_v, [idx], ones16)
        return 0

    lax.fori_loop(0, EPT // 16, acc_step, 0)
    pltpu.sync_copy(hist_v, hist_out.at[pl.ds(wid * CPW * V, CPW * V)])


def _sc_scatter_body(key_hbm, rin_hbm, ids_hbm, choff_hbm, sorted_out,
                     key_v, rin_v, ids_v, choff_v, pos_v, sem):
    wid = lax.axis_index("s") * 2 + lax.axis_index("c")
    base = wid * EPT
    pltpu.sync_copy(key_hbm.at[pl.ds(base, EPT)], key_v)
    pltpu.sync_copy(rin_hbm.at[pl.ds(base, EPT)], rin_v)
    pltpu.sync_copy(ids_hbm.at[pl.ds(base, EPT)], ids_v)
    pltpu.sync_copy(choff_hbm.at[pl.ds(wid * CPW * V, CPW * V)], choff_v)
    kbase = wid * CPW * V

    def pos_step(g, _):
        k16 = key_v[pl.ds(g * 16, 16)] - kbase
        off16 = plsc.load_gather(choff_v, [k16])
        pos_v[g // (C // 16), pl.ds((g % (C // 16)) * 16, 16)] = (
            off16 + rin_v[pl.ds(g * 16, 16)])
        return 0

    lax.fori_loop(0, EPT // 16, pos_step, 0)
    cps = [pltpu.async_copy(ids_v.at[pl.ds(j * LROW, LROW)],
                            sorted_out.at[pos_v.at[j]], sem)
           for j in range(CPW)]
    for cp in cps:
        cp.wait()


def _sc_gather_body(ptab_hbm, sorted_hbm, cnt_hbm, samp_hbm,
                    samp_out, cnt_out, ptab_v, id_v, cntg_v, rows_v, sem):
    wid = lax.axis_index("s") * 2 + lax.axis_index("c")
    base = wid * KB
    pltpu.sync_copy(ptab_hbm.at[pl.ds(base, KB)], ptab_v)
    cps = [pltpu.async_copy(sorted_hbm.at[ptab_v.at[pl.ds(j * LROW, LROW)]],
                            id_v.at[pl.ds(j * LROW, LROW)], sem)
           for j in range(KB // LROW)]
    for cp in cps:
        cp.wait()
    cps = [pltpu.async_copy(cnt_hbm.at[id_v.at[pl.ds(j * LROW, LROW)]],
                            cntg_v.at[pl.ds(j * LROW, LROW)], sem)
           for j in range(KB // LROW)]
    cps += [pltpu.async_copy(samp_hbm.at[id_v.at[pl.ds(j * LROW, LROW)]],
                             rows_v.at[pl.ds(j * LROW, LROW)], sem)
            for j in range(KB // LROW)]
    for cp in cps:
        cp.wait()
    pltpu.sync_copy(cntg_v, cnt_out.at[pl.ds(base, KB)])
    pltpu.sync_copy(rows_v, samp_out.at[pl.ds(base, KB)])


def kernel(uniq_samples, uniq_count):
    i32 = jnp.int32
    cpad = jnp.concatenate(
        [uniq_count, jnp.full((N_PAD - N_UNIQ,), PAD_VAL, dtype=i32)])
    cnt2 = cpad.reshape(B, C)

    mesh = plsc.VectorSubcoreMesh(
        core_axis_name="c", subcore_axis_name="s",
        num_cores=2, num_subcores=16)

    sc_hist = functools.partial(
        pl.kernel,
        out_type=jax.ShapeDtypeStruct((B * V,), i32),
        mesh=mesh,
        scratch_types=[pltpu.VMEM((EPT,), i32),
                       pltpu.VMEM((CPW * V,), i32)],
        compiler_params=pltpu.CompilerParams(needs_layout_passes=False),
    )(_sc_hist_body)

    hist = sc_hist(cpad).reshape(B, V)

    rin2, key2 = pl.pallas_call(
        _tc_rank_body,
        grid=(TC_STEPS,),
        in_specs=[pl.BlockSpec((CPB, C), lambda b: (b, 0))],
        out_specs=[
            pl.BlockSpec((CPB, C), lambda b: (b, 0)),
            pl.BlockSpec((CPB, C), lambda b: (b, 0)),
        ],
        out_shape=[
            jax.ShapeDtypeStruct((B, C), i32),
            jax.ShapeDtypeStruct((B, C), i32),
        ],
    )(cnt2)

    choff = pl.pallas_call(
        _tc_offs_body,
        out_shape=jax.ShapeDtypeStruct((B, V), i32),
    )(hist)

    sc_scatter = functools.partial(
        pl.kernel,
        out_type=jax.ShapeDtypeStruct((N_PAD,), i32),
        mesh=mesh,
        scratch_types=[
            pltpu.VMEM((EPT,), i32),
            pltpu.VMEM((EPT,), i32),
            pltpu.VMEM((EPT,), i32),
            pltpu.VMEM((CPW * V,), i32),
            pltpu.VMEM((CPW, LROW), i32),
            pltpu.SemaphoreType.DMA,
        ],
        compiler_params=pltpu.CompilerParams(needs_layout_passes=False),
    )(_sc_scatter_body)

    sorted_ids = sc_scatter(
        key2.reshape(-1), rin2.reshape(-1), jnp.asarray(_IDS),
        choff.reshape(-1))

    sc_gather = functools.partial(
        pl.kernel,
        out_type=(
            jax.ShapeDtypeStruct((2 * HALF, STATE_SIZE), jnp.float32),
            jax.ShapeDtypeStruct((2 * HALF,), i32),
        ),
        mesh=mesh,
        scratch_types=[
            pltpu.VMEM((KB,), i32),
            pltpu.VMEM((KB,), i32),
            pltpu.VMEM((KB,), i32),
            pltpu.VMEM((KB, STATE_SIZE), jnp.float32),
            pltpu.SemaphoreType.DMA,
        ],
    )(_sc_gather_body)

    out_samples, out_counts = sc_gather(
        jnp.asarray(_POS_TAB), sorted_ids, uniq_count, uniq_samples)
    return out_samples, out_counts


# speedup vs baseline: 116.8872x; 1.1581x over previous
import functools

import jax
import jax.numpy as jnp
import numpy as np
from jax import lax
from jax.experimental import pallas as pl
from jax.experimental.pallas import tpu as pltpu
from jax.experimental.pallas import tpu_sc as plsc

N_UNIQ = 100000
STATE_SIZE = 128
HALF = 8192
NUM_LEFT = N_UNIQ - HALF

V = 1024
C = 200
B = 512
N_PAD = B * C
PAD_VAL = 1023

NW = 32
LANES = 16
LROW = 128
CPW = LANES
EPT = CPW * C
KB = (2 * HALF) // NW


def _np_threefry2x32(k1, k2, x0, x1):
    def rotl(x, d):
        return ((x << np.uint32(d)) | (x >> np.uint32(32 - d))).astype(np.uint32)
    ks = [np.uint32(k1), np.uint32(k2),
          np.uint32(k1) ^ np.uint32(k2) ^ np.uint32(0x1BD11BDA)]
    rots = [(13, 15, 26, 6), (17, 29, 16, 24)]
    x0 = (x0 + ks[0]).astype(np.uint32)
    x1 = (x1 + ks[1]).astype(np.uint32)
    for i in range(5):
        for r in rots[i % 2]:
            x0 = (x0 + x1).astype(np.uint32)
            x1 = rotl(x1, r)
            x1 = x0 ^ x1
        x0 = (x0 + ks[(i + 1) % 3]).astype(np.uint32)
        x1 = (x1 + ks[(i + 2) % 3] + np.uint32(i + 1)).astype(np.uint32)
    return x0, x1


def _np_permutation(seed, n):
    key = np.array([np.uint32(np.uint64(seed) >> np.uint64(32)),
                    np.uint32(np.uint64(seed) & np.uint64(0xFFFFFFFF))])
    x = np.arange(n, dtype=np.int32)
    num_rounds = int(np.ceil(3 * np.log(max(1, n)) / np.log(2**32 - 1)))
    for _ in range(num_rounds):
        b1, b2 = _np_threefry2x32(key[0], key[1],
                                  np.zeros(2, np.uint32),
                                  np.arange(2, dtype=np.uint32))
        key, subkey = np.stack([b1, b2], 1)
        s1, s2 = _np_threefry2x32(subkey[0], subkey[1],
                                  np.zeros(n, np.uint32),
                                  np.arange(n, dtype=np.uint32))
        x = x[np.argsort(s1 ^ s2, kind="stable")]
    return x


_PERM = _np_permutation(12345, NUM_LEFT)[:HALF]
_POS_TAB = np.concatenate([NUM_LEFT + np.arange(HALF, dtype=np.int32),
                           _PERM.astype(np.int32)])


def _sc_histrank_body(cnt_hbm, hist_out, rin_out, key_out,
                      cnt_v, hist_v, rin_v, key_v):
    wid = lax.axis_index("s") * 2 + lax.axis_index("c")
    base = wid * EPT
    pltpu.sync_copy(cnt_hbm.at[pl.ds(base, EPT)], cnt_v)
    iota16 = lax.broadcasted_iota(jnp.int32, (LANES,), 0)
    zeros16 = jnp.zeros((LANES,), jnp.int32)
    ones16 = jnp.ones((LANES,), jnp.int32)
    kbase = wid * CPW * V

    def zero_step(i, _):
        hist_v[pl.ds(i * LANES, LANES)] = zeros16
        return 0

    lax.fori_loop(0, CPW * V // LANES, zero_step, 0)

    def elem_step(e, _):
        idx16 = iota16 * C + e
        c16 = plsc.load_gather(cnt_v, [idx16])
        h_idx = iota16 * V + c16
        old = plsc.load_gather(hist_v, [h_idx])
        plsc.store_scatter(rin_v, [idx16], old)
        plsc.store_scatter(key_v, [idx16], h_idx + kbase)
        plsc.addupdate_scatter(hist_v, [h_idx], ones16)
        return 0

    lax.fori_loop(0, C, elem_step, 0)
    pltpu.sync_copy(hist_v, hist_out.at[pl.ds(wid * CPW * V, CPW * V)])
    pltpu.sync_copy(rin_v, rin_out.at[pl.ds(base, EPT)])
    pltpu.sync_copy(key_v, key_out.at[pl.ds(base, EPT)])


def _tc_offs_body(hist_ref, out_ref):
    hist = hist_ref[...].astype(jnp.float32)
    tri = (lax.broadcasted_iota(jnp.int32, (B, B), 1)
           < lax.broadcasted_iota(jnp.int32, (B, B), 0)).astype(jnp.float32)
    colcum = jnp.dot(tri, hist, preferred_element_type=jnp.float32)
    total = jnp.sum(hist, axis=0, keepdims=True)
    mv = (lax.broadcasted_iota(jnp.int32, (V, V), 0)
          < lax.broadcasted_iota(jnp.int32, (V, V), 1)).astype(jnp.float32)
    glob = jnp.dot(total, mv, preferred_element_type=jnp.float32)
    out_ref[...] = (colcum + glob).astype(jnp.int32)


def _sc_scatter_body(key_hbm, rin_hbm, choff_hbm, sorted_out,
                     key_v, rin_v, choff_v, ids_v, pos_v, sem):
    wid = lax.axis_index("s") * 2 + lax.axis_index("c")
    base = wid * EPT
    pltpu.sync_copy(key_hbm.at[pl.ds(base, EPT)], key_v)
    pltpu.sync_copy(rin_hbm.at[pl.ds(base, EPT)], rin_v)
    pltpu.sync_copy(choff_hbm.at[pl.ds(wid * CPW * V, CPW * V)], choff_v)
    iota16 = lax.broadcasted_iota(jnp.int32, (LANES,), 0)
    kbase = wid * CPW * V
    gpr = LROW // LANES

    def pos_step(g, _):
        k16 = key_v[pl.ds(g * LANES, LANES)] - kbase
        off16 = plsc.load_gather(choff_v, [k16])
        pos_v[g // gpr, pl.ds((g % gpr) * LANES, LANES)] = (
            off16 + rin_v[pl.ds(g * LANES, LANES)])
        ids_v[pl.ds(g * LANES, LANES)] = base + g * LANES + iota16
        return 0

    lax.fori_loop(0, EPT // LANES, pos_step, 0)
    cps = [pltpu.async_copy(ids_v.at[pl.ds(j * LROW, LROW)],
                            sorted_out.at[pos_v.at[j]], sem)
           for j in range(EPT // LROW)]
    for cp in cps:
        cp.wait()


def _sc_gather_body(ptab_hbm, sorted_hbm, cnt_hbm, samp_hbm,
                    samp_out, cnt_out, ptab_v, id_v, cntg_v, rows_v, sem):
    wid = lax.axis_index("s") * 2 + lax.axis_index("c")
    base = wid * KB
    pltpu.sync_copy(ptab_hbm.at[pl.ds(base, KB)], ptab_v)
    cps = [pltpu.async_copy(sorted_hbm.at[ptab_v.at[pl.ds(j * LROW, LROW)]],
                            id_v.at[pl.ds(j * LROW, LROW)], sem)
           for j in range(KB // LROW)]
    for cp in cps:
        cp.wait()
    cps = [pltpu.async_copy(cnt_hbm.at[id_v.at[pl.ds(j * LROW, LROW)]],
                            cntg_v.at[pl.ds(j * LROW, LROW)], sem)
           for j in range(KB // LROW)]
    cps += [pltpu.async_copy(samp_hbm.at[id_v.at[pl.ds(j * LROW, LROW)]],
                             rows_v.at[pl.ds(j * LROW, LROW)], sem)
            for j in range(KB // LROW)]
    for cp in cps:
        cp.wait()
    pltpu.sync_copy(cntg_v, cnt_out.at[pl.ds(base, KB)])
    pltpu.sync_copy(rows_v, samp_out.at[pl.ds(base, KB)])


def kernel(uniq_samples, uniq_count):
    i32 = jnp.int32
    cpad = jnp.concatenate(
        [uniq_count, jnp.full((N_PAD - N_UNIQ,), PAD_VAL, dtype=i32)])

    mesh = plsc.VectorSubcoreMesh(
        core_axis_name="c", subcore_axis_name="s",
        num_cores=2, num_subcores=16)

    sc_histrank = functools.partial(
        pl.kernel,
        out_type=(
            jax.ShapeDtypeStruct((B * V,), i32),
            jax.ShapeDtypeStruct((N_PAD,), i32),
            jax.ShapeDtypeStruct((N_PAD,), i32),
        ),
        mesh=mesh,
        scratch_types=[pltpu.VMEM((EPT,), i32),
                       pltpu.VMEM((CPW * V,), i32),
                       pltpu.VMEM((EPT,), i32),
                       pltpu.VMEM((EPT,), i32)],
        compiler_params=pltpu.CompilerParams(needs_layout_passes=False),
    )(_sc_histrank_body)

    hist, rin, key = sc_histrank(cpad)

    choff = pl.pallas_call(
        _tc_offs_body,
        out_shape=jax.ShapeDtypeStruct((B, V), i32),
    )(hist.reshape(B, V))

    sc_scatter = functools.partial(
        pl.kernel,
        out_type=jax.ShapeDtypeStruct((N_PAD,), i32),
        mesh=mesh,
        scratch_types=[
            pltpu.VMEM((EPT,), i32),
            pltpu.VMEM((EPT,), i32),
            pltpu.VMEM((CPW * V,), i32),
            pltpu.VMEM((EPT,), i32),
            pltpu.VMEM((EPT // LROW, LROW), i32),
            pltpu.SemaphoreType.DMA,
        ],
        compiler_params=pltpu.CompilerParams(needs_layout_passes=False),
    )(_sc_scatter_body)

    sorted_ids = sc_scatter(key, rin, choff.reshape(-1))

    sc_gather = functools.partial(
        pl.kernel,
        out_type=(
            jax.ShapeDtypeStruct((2 * HALF, STATE_SIZE), jnp.float32),
            jax.ShapeDtypeStruct((2 * HALF,), i32),
        ),
        mesh=mesh,
        scratch_types=[
            pltpu.VMEM((KB,), i32),
            pltpu.VMEM((KB,), i32),
            pltpu.VMEM((KB,), i32),
            pltpu.VMEM((KB, STATE_SIZE), jnp.float32),
            pltpu.SemaphoreType.DMA,
        ],
    )(_sc_gather_body)

    out_samples, out_counts = sc_gather(
        jnp.asarray(_POS_TAB), sorted_ids, uniq_count, uniq_samples)
    return out_samples, out_counts
